# baseline (device time: 34826 ns/iter reference)
import jax
import jax.numpy as jnp
from jax import lax
from jax.experimental import pallas as pl
from jax.experimental.pallas import tpu as pltpu

N_DEV = 4
B_LOC = 2
SQ = 256
SKV = 256
HQ = 16
HQ_LOC = 4
DH = 64
D_MODEL = 512
HD = HQ * DH
HD_LOC = HQ_LOC * DH
WINDOW = 128


def _body(x_ref, wq_ref, k_hbm, v_hbm, wo_ref, out_ref,
          xb_ref, wqb_ref, wob_ref, kf_ref, vf_ref, kb_ref, vb_ref,
          cwq, cwo, ctx_ref, local_sems, swq, rwq, swo, rwo):
    my = lax.axis_index("i")

    cp_k = pltpu.make_async_copy(
        k_hbm.at[pl.ds(my * B_LOC, B_LOC)], kf_ref, local_sems.at[0])
    cp_v = pltpu.make_async_copy(
        v_hbm.at[pl.ds(my * B_LOC, B_LOC)], vf_ref, local_sems.at[1])
    cp_k.start()
    cp_v.start()

    wqb_ref[...] = wq_ref[...].astype(jnp.bfloat16)
    wob_ref[...] = wo_ref[...].astype(jnp.bfloat16)

    bar = pltpu.get_barrier_semaphore()
    for r in range(1, N_DEV):
        peer = lax.rem(my + r, N_DEV)
        pl.semaphore_signal(bar, inc=1, device_id=(peer,),
                            device_id_type=pl.DeviceIdType.MESH)
    pl.semaphore_wait(bar, N_DEV - 1)

    sends = []
    for r in range(1, N_DEV):
        peer = lax.rem(my + r, N_DEV)
        rd_q = pltpu.make_async_remote_copy(
            src_ref=wqb_ref, dst_ref=cwq.at[r - 1],
            send_sem=swq.at[r - 1], recv_sem=rwq.at[r - 1],
            device_id=(peer,), device_id_type=pl.DeviceIdType.MESH)
        rd_o = pltpu.make_async_remote_copy(
            src_ref=wob_ref, dst_ref=cwo.at[r - 1],
            send_sem=swo.at[r - 1], recv_sem=rwo.at[r - 1],
            device_id=(peer,), device_id_type=pl.DeviceIdType.MESH)
        rd_q.start()
        rd_o.start()
        sends.append((rd_q, rd_o))

    xb_ref[...] = x_ref[...].astype(jnp.bfloat16)
    cp_k.wait()
    cp_v.wait()
    kb_ref[...] = kf_ref[...].astype(jnp.bfloat16)
    vb_ref[...] = vf_ref[...].astype(jnp.bfloat16)

    qi = lax.broadcasted_iota(jnp.int32, (SQ, SKV), 0)
    ki = lax.broadcasted_iota(jnp.int32, (SQ, SKV), 1)
    mask = jnp.abs(qi - ki) <= WINDOW

    def compute_chunk(origin, wq_c, wo_c, first):
        q = jnp.dot(xb_ref[...], wq_c, preferred_element_type=jnp.float32)
        q = (q * 0.125).astype(jnp.bfloat16)
        for b in range(B_LOC):
            kb = kb_ref[b, :, pl.ds(origin * HD_LOC, HD_LOC)]
            vb = vb_ref[b, :, pl.ds(origin * HD_LOC, HD_LOC)]
            for h in range(HQ_LOC):
                qh = q[b * SQ:(b + 1) * SQ, h * DH:(h + 1) * DH]
                s = lax.dot_general(
                    qh, kb[:, h * DH:(h + 1) * DH],
                    (((1,), (1,)), ((), ())),
                    preferred_element_type=jnp.float32)
                s = jnp.where(mask, s, -1e9)
                s = s - jnp.max(s, axis=1, keepdims=True)
                w = jnp.exp(s)
                w = (w / jnp.sum(w, axis=1, keepdims=True)).astype(jnp.bfloat16)
                ctx = jnp.dot(w, vb[:, h * DH:(h + 1) * DH],
                              preferred_element_type=jnp.float32)
                ctx_ref[b * SQ:(b + 1) * SQ, h * DH:(h + 1) * DH] = (
                    ctx.astype(jnp.bfloat16))
        part = jnp.dot(ctx_ref[...], wo_c, preferred_element_type=jnp.float32)
        if first:
            out_ref[...] = part
        else:
            out_ref[...] = out_ref[...] + part

    compute_chunk(my, wqb_ref[...], wob_ref[...], first=True)
    for r in (1, 3, 2):
        rd_q, rd_o = sends[r - 1]
        rd_q.wait_recv()
        rd_o.wait_recv()
        origin = lax.rem(my - r + N_DEV, N_DEV)
        compute_chunk(origin, cwq[r - 1], cwo[r - 1], first=False)

    for rd_q, rd_o in sends:
        rd_q.wait_send()
        rd_o.wait_send()


def kernel(x, Wq, K_ext, V_ext, Wo):
    x2d = x.reshape(B_LOC * SQ, D_MODEL)
    K3 = K_ext.reshape(N_DEV * B_LOC, SKV, HD)
    V3 = V_ext.reshape(N_DEV * B_LOC, SKV, HD)

    out2d = pl.pallas_call(
        _body,
        out_shape=jax.ShapeDtypeStruct((B_LOC * SQ, D_MODEL), jnp.float32),
        in_specs=[
            pl.BlockSpec(memory_space=pltpu.VMEM),
            pl.BlockSpec(memory_space=pltpu.VMEM),
            pl.BlockSpec(memory_space=pl.ANY),
            pl.BlockSpec(memory_space=pl.ANY),
            pl.BlockSpec(memory_space=pltpu.VMEM),
        ],
        out_specs=pl.BlockSpec(memory_space=pltpu.VMEM),
        scratch_shapes=[
            pltpu.VMEM((B_LOC * SQ, D_MODEL), jnp.bfloat16),
            pltpu.VMEM((D_MODEL, HD_LOC), jnp.bfloat16),
            pltpu.VMEM((HD_LOC, D_MODEL), jnp.bfloat16),
            pltpu.VMEM((B_LOC, SKV, HD), jnp.float32),
            pltpu.VMEM((B_LOC, SKV, HD), jnp.float32),
            pltpu.VMEM((B_LOC, SKV, HD), jnp.bfloat16),
            pltpu.VMEM((B_LOC, SKV, HD), jnp.bfloat16),
            pltpu.VMEM((N_DEV - 1, D_MODEL, HD_LOC), jnp.bfloat16),
            pltpu.VMEM((N_DEV - 1, HD_LOC, D_MODEL), jnp.bfloat16),
            pltpu.VMEM((B_LOC * SQ, HD_LOC), jnp.bfloat16),
            pltpu.SemaphoreType.DMA((2,)),
            pltpu.SemaphoreType.DMA((N_DEV - 1,)),
            pltpu.SemaphoreType.DMA((N_DEV - 1,)),
            pltpu.SemaphoreType.DMA((N_DEV - 1,)),
            pltpu.SemaphoreType.DMA((N_DEV - 1,)),
        ],
        compiler_params=pltpu.CompilerParams(collective_id=0),
    )(x2d, Wq, K3, V3, Wo)
    return out2d.reshape(B_LOC, SQ, D_MODEL)
